# Initial kernel scaffold; baseline (speedup 1.0000x reference)
import jax
import jax.numpy as jnp
from jax.experimental import pallas as pl


def kernel(x_user, adj_user, x_item, adj_item, W_user, b_user, W_item, b_item):
    raise NotImplementedError("write your pallas kernel here")


# speedup vs baseline: 58.2654x; 58.2654x over previous
import functools

import jax
import jax.numpy as jnp
from jax import lax
from jax.experimental import pallas as pl
from jax.experimental.pallas import tpu as pltpu
from jax.experimental.pallas import tpu_sc as plsc

NP = 51200
RPT = NP // 16
NT = 16
NC = 2
CH = 128
K = 8
TW = 16


def _mesh():
    return plsc.VectorSubcoreMesh(core_axis_name="c", subcore_axis_name="s")


def _deg_kernel_body(nch):
    def body(dst_hbm, zeros_hbm, deg_out, slab, ones, acc, sem):
        c = lax.axis_index("c")
        s = lax.axis_index("s")
        wid = c * NT + s
        pltpu.sync_copy(dst_hbm.at[wid], slab)

        def fill(i, carry):
            ones[pl.ds(i * 16, 16)] = jnp.full((16,), 1.0, jnp.float32)
            return carry
        lax.fori_loop(0, CH // 16, fill, 0)

        pltpu.sync_copy(zeros_hbm.at[pl.ds(s * RPT, RPT)],
                        acc.at[pl.ds(s * RPT, RPT)])
        plsc.subcore_barrier()

        def group(gi, carry):
            descs = []
            for b in range(K):
                d = pltpu.async_copy(ones, acc.at[slab.at[gi * K + b]],
                                     sem, add=True)
                descs.append(d)
            for d in descs:
                d.wait()
            return carry
        lax.fori_loop(0, nch // K, group, 0)

        plsc.subcore_barrier()
        pltpu.sync_copy(acc.at[pl.ds(s * RPT, RPT)],
                        deg_out.at[pl.ds(c * NP + s * RPT, RPT)])
    return body


def _edge_kernel_body(nch):
    ngroups = nch // K
    assert ngroups % 2 == 0

    def body(idx_hbm, table_hbm, zeros_hbm, s_out, ibuf, rows, acc,
             isem, gsem, ssem):
        c = lax.axis_index("c")
        s = lax.axis_index("s")
        wid = c * NT + s
        pltpu.sync_copy(zeros_hbm.at[pl.ds(s * RPT, RPT)],
                        acc.at[pl.ds(s * RPT, RPT)])
        plsc.subcore_barrier()

        pltpu.async_copy(idx_hbm.at[wid, pl.ds(0, K)], ibuf.at[0], isem)

        def outer(g2, carry):
            for slot in range(2):
                g = g2 * 2 + slot
                pltpu.make_async_copy(idx_hbm.at[wid, pl.ds(g * K, K)],
                                      ibuf.at[slot], isem).wait()

                @pl.when(g + 1 < ngroups)
                def _():
                    pltpu.async_copy(
                        idx_hbm.at[wid, pl.ds((g + 1) * K, K)],
                        ibuf.at[1 - slot], isem)

                gdescs = []
                for k in range(K):
                    d = pltpu.async_copy(table_hbm.at[ibuf.at[slot, k, 0]],
                                         rows.at[k], gsem)
                    gdescs.append(d)
                sdescs = []
                for k in range(K):
                    gdescs[k].wait()
                    d = pltpu.async_copy(rows.at[k],
                                         acc.at[ibuf.at[slot, k, 1]],
                                         ssem, add=True)
                    sdescs.append(d)
                for d in sdescs:
                    d.wait()
            return carry
        lax.fori_loop(0, ngroups // 2, outer, 0)

        plsc.subcore_barrier()
        pltpu.sync_copy(acc.at[pl.ds(s * RPT, RPT)],
                        s_out.at[pl.ds(c * NP + s * RPT, RPT)])
    return body


def _scale_tc_kernel(x_ref, w_ref, deg_ref, g_ref):
    h = jnp.dot(x_ref[0], w_ref[0], preferred_element_type=jnp.float32)
    c = pl.program_id(0)
    dinv = lax.rsqrt(deg_ref[c] + 1.0)
    g = h * dinv[:, None]
    br = h.shape[0]
    g_ref[0] = jnp.concatenate(
        [g, dinv[:, None], jnp.zeros((br, TW - 9), jnp.float32)], axis=1)


def _decode_tc_kernel(g_ref, s_ref, b_ref, o_ref):
    def branch(c):
        g = g_ref[c]
        sa = s_ref[c]
        dinv = g[:, 8:9]
        return jnp.tanh(dinv * (sa[:, 0:8] + g[:, 0:8]) + b_ref[c][None, :])
    val = jnp.sum(branch(0) * branch(1), axis=1)
    o_ref[...] = val.reshape(o_ref.shape)


def kernel(x_user, adj_user, x_item, adj_item, W_user, b_user, W_item, b_item):
    n, d_in = x_user.shape
    e = adj_user.shape[1]
    nch = -(-e // (NT * CH))
    nch = -(-nch // (2 * K)) * (2 * K)
    ept = nch * CH
    pad = NT * ept - e

    def prep(adj, offset):
        src = adj[0].astype(jnp.int32) + offset
        dst = adj[1].astype(jnp.int32)
        src = jnp.concatenate(
            [src, jnp.full((pad,), offset + n, jnp.int32)])
        dst = jnp.concatenate([dst, jnp.full((pad,), n, jnp.int32)])
        return src.reshape(NT, nch, CH), dst.reshape(NT, nch, CH)

    su, du = prep(adj_user, 0)
    si, di = prep(adj_item, NP)
    src_all = jnp.concatenate([su, si], axis=0)
    dst_all = jnp.concatenate([du, di], axis=0)
    idx_all = jnp.stack([src_all, dst_all], axis=2)

    zeros1 = jnp.zeros((NP,), jnp.float32)
    zeros2 = jnp.zeros((NP, TW), jnp.float32)

    deg_kernel = pl.kernel(
        _deg_kernel_body(nch),
        out_type=jax.ShapeDtypeStruct((NC * NP,), jnp.float32),
        mesh=_mesh(),
        scratch_types=[
            pltpu.VMEM((nch, CH), jnp.int32),
            pltpu.VMEM((CH,), jnp.float32),
            pltpu.VMEM_SHARED((NP,), jnp.float32),
            pltpu.SemaphoreType.DMA,
        ],
        compiler_params=pltpu.CompilerParams(use_tc_tiling_on_sc=False),
    )
    deg = deg_kernel(dst_all, zeros1)

    rowpad = jnp.zeros((NP - n, d_in), jnp.float32)
    xp = jnp.stack([jnp.concatenate([x_user, rowpad], axis=0),
                    jnp.concatenate([x_item, rowpad], axis=0)])
    w_all = jnp.stack([W_user, W_item])
    deg2 = deg.reshape(NC, NP)

    BR = 2048
    nb = NP // BR
    g_all = pl.pallas_call(
        _scale_tc_kernel,
        grid=(NC, nb),
        in_specs=[
            pl.BlockSpec((1, BR, d_in), lambda c, i: (c, i, 0)),
            pl.BlockSpec((1, d_in, 8), lambda c, i: (c, 0, 0)),
            pl.BlockSpec((NC, BR), lambda c, i: (0, i)),
        ],
        out_specs=pl.BlockSpec((1, BR, TW), lambda c, i: (c, i, 0)),
        out_shape=jax.ShapeDtypeStruct((NC, NP, TW), jnp.float32),
    )(xp, w_all, deg2)

    table = g_all.reshape(NC * NP, TW)

    edge_kernel = pl.kernel(
        _edge_kernel_body(nch),
        out_type=jax.ShapeDtypeStruct((NC * NP, TW), jnp.float32),
        mesh=_mesh(),
        scratch_types=[
            pltpu.VMEM((2, K, 2, CH), jnp.int32),
            pltpu.VMEM((K, CH, TW), jnp.float32),
            pltpu.VMEM_SHARED((NP, TW), jnp.float32),
            pltpu.SemaphoreType.DMA,
            pltpu.SemaphoreType.DMA,
            pltpu.SemaphoreType.DMA,
        ],
        compiler_params=pltpu.CompilerParams(use_tc_tiling_on_sc=False),
    )
    s_acc = edge_kernel(idx_all, table, zeros2)

    b_all = jnp.stack([b_user, b_item])
    score = pl.pallas_call(
        _decode_tc_kernel,
        grid=(nb,),
        in_specs=[
            pl.BlockSpec((NC, BR, TW), lambda i: (0, i, 0)),
            pl.BlockSpec((NC, BR, TW), lambda i: (0, i, 0)),
            pl.BlockSpec((NC, 8), lambda i: (0, 0)),
        ],
        out_specs=pl.BlockSpec((BR // 128, 128), lambda i: (i, 0)),
        out_shape=jax.ShapeDtypeStruct((NP // 128, 128), jnp.float32),
    )(g_all, s_acc.reshape(NC, NP, TW), b_all)

    return score.reshape(NP)[:n]
